# baseline (speedup 1.0000x reference)
import jax
import jax.numpy as jnp
from jax import lax
from jax.experimental import pallas as pl
from jax.experimental.pallas import tpu as pltpu
from jax.experimental.pallas import tpu_sc as plsc

SCORE_THR = 0.2
IOU_THR = 0.7
TOPK = 10
N_BOXES = 5000
LANES = 16
NBLK = 20
ROWS = NBLK * LANES
PAD = ROWS * LANES
NROUND = 40
NVISIT = 128
NEG = -jnp.inf
BIG = 2**30


def _nms_body(scores_h, box_h, out_h, s_v, c_s, tmp_v, bm_v, br_v, out_v,
              kept_v, st, sem):
    cid = lax.axis_index("c")
    sid = lax.axis_index("s")

    lanes = lax.iota(jnp.int32, LANES)

    def _shuf(v, d):
        return v.at[lanes ^ d].get(mode="promise_in_bounds")

    def _allmax(v):
        for d in (1, 2, 4, 8):
            v = jnp.maximum(v, _shuf(v, d))
        return v

    def _allmin(v):
        for d in (1, 2, 4, 8):
            v = jnp.minimum(v, _shuf(v, d))
        return v

    def _blockmax(b):
        m = s_v[b * LANES, :]
        a = jnp.zeros((LANES,), jnp.int32)
        for j in range(1, LANES):
            v = s_v[b * LANES + j, :]
            upd = v > m
            m = jnp.where(upd, v, m)
            a = jnp.where(upd, j, a)
        return m, a

    @pl.when((cid == 0) & (sid == 0))
    def _():
        pltpu.sync_copy(scores_h, s_v)
        pltpu.sync_copy(box_h, c_s)
        z16 = jnp.zeros((LANES,), jnp.float32)
        for r in range(16):
            out_v[r, :] = z16
        for r in range(6):
            kept_v[r, :] = z16
        zi16 = jnp.zeros((LANES,), jnp.int32)
        st[0, :] = zi16
        st[1, :] = zi16

        def build(b, carry):
            m, a = _blockmax(b)
            bm_v[b, :] = m
            br_v[b, :] = a
            return carry

        lax.fori_loop(0, NBLK, build, jnp.int32(0), unroll=4)

        zero = jnp.float32(0.0)

        def visit(_, carry):
            @pl.when(st[0, :][0] == 0)
            def _():
                def scan_blk(b, mc):
                    cmv, cbv = mc
                    v = bm_v[b, :]
                    upd = v > cmv
                    return (jnp.where(upd, v, cmv),
                            jnp.where(upd, b, cbv))

                cmv, cbv = lax.fori_loop(
                    0, NBLK, scan_blk,
                    (jnp.full((LANES,), NEG, jnp.float32),
                     jnp.zeros((LANES,), jnp.int32)), unroll=5)
                mxv = _allmax(cmv)
                b0 = _allmin(jnp.where(cmv == mxv, cbv, jnp.int32(BIG)))[0]

                i16 = ((b0 * LANES + br_v[b0, :]) * LANES + lanes)
                idx = _allmin(
                    jnp.where(bm_v[b0, :] == mxv, i16, jnp.int32(BIG)))[0]

                r = lax.shift_right_logical(idx, 4)
                lm = lanes == (idx & (LANES - 1))

                s_v[r, :] = jnp.where(lm, NEG, s_v[r, :])
                br = lax.shift_right_logical(idx, 2)
                cp = pltpu.make_async_copy(c_s.at[pl.ds(br, 1)], tmp_v, sem)
                cp.start()
                m, a = _blockmax(b0)
                bm_v[b0, :] = m
                br_v[b0, :] = a
                cp.wait()
                base = (idx & 3) * 4
                rot = tmp_v[0, :].at[(lanes + base) & (LANES - 1)].get(
                    mode="promise_in_bounds")
                bx1 = rot[0]
                by1 = rot[1]
                bx2 = rot[2]
                by2 = rot[3]
                barea = (bx2 - bx1) * (by2 - by1)

                kv = st[1, :]
                w = jnp.maximum(
                    jnp.minimum(kept_v[2, :], bx2)
                    - jnp.maximum(kept_v[0, :], bx1), zero)
                h = jnp.maximum(
                    jnp.minimum(kept_v[3, :], by2)
                    - jnp.maximum(kept_v[1, :], by1), zero)
                inter = w * h
                union = kept_v[4, :] + barea - inter
                iou = inter / jnp.maximum(union, jnp.float32(1e-9))
                ov = jnp.where((iou > IOU_THR) & (lanes < kv),
                               jnp.int32(1), jnp.int32(0))
                overl_iv = jnp.minimum(_allmax(ov), jnp.int32(1))
                below_iv = jnp.where(mxv < SCORE_THR,
                                     jnp.int32(1), jnp.int32(0))

                row = (jnp.where(lanes == 0, bx1, zero)
                       + jnp.where(lanes == 1, by1, zero)
                       + jnp.where(lanes == 2, bx2, zero)
                       + jnp.where(lanes == 3, by2, zero)
                       + jnp.where(lanes == 4, mxv, zero))
                keep_iv = ((jnp.int32(1) - overl_iv)
                           * (jnp.int32(1) - below_iv))
                kcnt = kv[0]
                keepv = lanes < keep_iv * LANES
                out_v[kcnt, :] = jnp.where(keepv, row, out_v[kcnt, :])
                sel = keepv & (lanes == kv)
                kept_v[0, :] = jnp.where(sel, bx1, kept_v[0, :])
                kept_v[1, :] = jnp.where(sel, by1, kept_v[1, :])
                kept_v[2, :] = jnp.where(sel, bx2, kept_v[2, :])
                kept_v[3, :] = jnp.where(sel, by2, kept_v[3, :])
                kept_v[4, :] = jnp.where(sel, barea, kept_v[4, :])
                nkv = kv + keep_iv
                st[1, :] = nkv
                ge_iv = jnp.where(nkv >= TOPK, jnp.int32(1), jnp.int32(0))
                st[0, :] = jnp.maximum(below_iv, ge_iv)
            return carry

        def rnd(_, carry):
            @pl.when(st[0, :][0] == 0)
            def _():
                lax.fori_loop(0, NVISIT, visit, jnp.int32(0))
            return carry

        lax.fori_loop(0, NROUND, rnd, jnp.int32(0))
        pltpu.sync_copy(out_v, out_h)


_sc_nms = pl.kernel(
    _nms_body,
    out_type=jax.ShapeDtypeStruct((16, LANES), jnp.float32),
    mesh=plsc.VectorSubcoreMesh(core_axis_name="c", subcore_axis_name="s"),
    scratch_types=[
        pltpu.VMEM((ROWS, LANES), jnp.float32),
        pltpu.VMEM_SHARED((4 * ROWS, LANES), jnp.float32),
        pltpu.VMEM((1, LANES), jnp.float32),
        pltpu.VMEM((NBLK, LANES), jnp.float32),
        pltpu.VMEM((NBLK, LANES), jnp.int32),
        pltpu.VMEM((16, LANES), jnp.float32),
        pltpu.VMEM((8, LANES), jnp.float32),
        pltpu.VMEM((2, LANES), jnp.int32),
        pltpu.SemaphoreType.DMA,
    ],
)


@jax.jit
def kernel(boxes, scores):
    npad = PAD - N_BOXES
    s = jnp.concatenate(
        [scores, jnp.full((npad,), NEG, jnp.float32)]).reshape(ROWS, LANES)
    b = jnp.concatenate([boxes, jnp.zeros((npad, 4), jnp.float32)])
    box = b.reshape(4 * ROWS, LANES)
    out = _sc_nms(s, box)
    return out[:TOPK, :5]


# speedup vs baseline: 1.0658x; 1.0658x over previous
import jax
import jax.numpy as jnp
from jax import lax
from jax.experimental import pallas as pl
from jax.experimental.pallas import tpu as pltpu
from jax.experimental.pallas import tpu_sc as plsc

SCORE_THR = 0.2
IOU_THR = 0.7
TOPK = 10
N_BOXES = 5000
LANES = 16
NBLK = 20
ROWS = NBLK * LANES
PAD = ROWS * LANES
NROUND = 20
NVISIT = 256
NEG = -jnp.inf
BIG = 2**30


def _nms_body(scores_h, box_h, out_h, s_v, c_s, tmp_v, bm_v, br_v, out_v,
              kept_v, st, sem):
    cid = lax.axis_index("c")
    sid = lax.axis_index("s")

    lanes = lax.iota(jnp.int32, LANES)

    def _shuf(v, d):
        return v.at[lanes ^ d].get(mode="promise_in_bounds")

    def _allmax(v):
        for d in (1, 2, 4, 8):
            v = jnp.maximum(v, _shuf(v, d))
        return v

    def _allmin(v):
        for d in (1, 2, 4, 8):
            v = jnp.minimum(v, _shuf(v, d))
        return v

    def _blockmax(b):
        m = s_v[b * LANES, :]
        a = jnp.zeros((LANES,), jnp.int32)
        for j in range(1, LANES):
            v = s_v[b * LANES + j, :]
            upd = v > m
            m = jnp.where(upd, v, m)
            a = jnp.where(upd, j, a)
        return m, a

    @pl.when((cid == 0) & (sid == 0))
    def _():
        pltpu.sync_copy(scores_h, s_v)
        pltpu.sync_copy(box_h, c_s)
        z16 = jnp.zeros((LANES,), jnp.float32)
        for r in range(16):
            out_v[r, :] = z16
        for r in range(6):
            kept_v[r, :] = z16
        zi16 = jnp.zeros((LANES,), jnp.int32)
        st[0, :] = zi16
        st[1, :] = zi16

        def build(b, carry):
            m, a = _blockmax(b)
            bm_v[b, :] = m
            br_v[b, :] = a
            return carry

        lax.fori_loop(0, NBLK, build, jnp.int32(0), unroll=4)

        zero = jnp.float32(0.0)

        def visit(_, carry):
            @pl.when(st[0, :][0] == 0)
            def _():
                def scan_blk(b, mc):
                    cmv, cbv = mc
                    v = bm_v[b, :]
                    upd = v > cmv
                    return (jnp.where(upd, v, cmv),
                            jnp.where(upd, b, cbv))

                cmv, cbv = lax.fori_loop(
                    0, NBLK, scan_blk,
                    (jnp.full((LANES,), NEG, jnp.float32),
                     jnp.zeros((LANES,), jnp.int32)), unroll=5)
                mxv = _allmax(cmv)
                b0 = _allmin(jnp.where(cmv == mxv, cbv, jnp.int32(BIG)))[0]

                i16 = ((b0 * LANES + br_v[b0, :]) * LANES + lanes)
                idx = _allmin(
                    jnp.where(bm_v[b0, :] == mxv, i16, jnp.int32(BIG)))[0]

                r = lax.shift_right_logical(idx, 4)
                lm = lanes == (idx & (LANES - 1))

                s_v[r, :] = jnp.where(lm, NEG, s_v[r, :])
                br = lax.shift_right_logical(idx, 2)
                cp = pltpu.make_async_copy(c_s.at[pl.ds(br, 1)], tmp_v, sem)
                cp.start()
                m, a = _blockmax(b0)
                bm_v[b0, :] = m
                br_v[b0, :] = a
                cp.wait()
                base = (idx & 3) * 4
                rot = tmp_v[0, :].at[(lanes + base) & (LANES - 1)].get(
                    mode="promise_in_bounds")
                bx1 = rot[0]
                by1 = rot[1]
                bx2 = rot[2]
                by2 = rot[3]
                barea = (bx2 - bx1) * (by2 - by1)

                kv = st[1, :]
                w = jnp.maximum(
                    jnp.minimum(kept_v[2, :], bx2)
                    - jnp.maximum(kept_v[0, :], bx1), zero)
                h = jnp.maximum(
                    jnp.minimum(kept_v[3, :], by2)
                    - jnp.maximum(kept_v[1, :], by1), zero)
                inter = w * h
                union = kept_v[4, :] + barea - inter
                iou = inter / jnp.maximum(union, jnp.float32(1e-9))
                ov = jnp.where((iou > IOU_THR) & (lanes < kv),
                               jnp.int32(1), jnp.int32(0))
                overl_iv = jnp.minimum(_allmax(ov), jnp.int32(1))
                below_iv = jnp.where(mxv < SCORE_THR,
                                     jnp.int32(1), jnp.int32(0))

                row = (jnp.where(lanes == 0, bx1, zero)
                       + jnp.where(lanes == 1, by1, zero)
                       + jnp.where(lanes == 2, bx2, zero)
                       + jnp.where(lanes == 3, by2, zero)
                       + jnp.where(lanes == 4, mxv, zero))
                keep_iv = ((jnp.int32(1) - overl_iv)
                           * (jnp.int32(1) - below_iv))
                kcnt = kv[0]
                keepv = lanes < keep_iv * LANES
                out_v[kcnt, :] = jnp.where(keepv, row, out_v[kcnt, :])
                sel = keepv & (lanes == kv)
                kept_v[0, :] = jnp.where(sel, bx1, kept_v[0, :])
                kept_v[1, :] = jnp.where(sel, by1, kept_v[1, :])
                kept_v[2, :] = jnp.where(sel, bx2, kept_v[2, :])
                kept_v[3, :] = jnp.where(sel, by2, kept_v[3, :])
                kept_v[4, :] = jnp.where(sel, barea, kept_v[4, :])
                nkv = kv + keep_iv
                st[1, :] = nkv
                ge_iv = jnp.where(nkv >= TOPK, jnp.int32(1), jnp.int32(0))
                st[0, :] = jnp.maximum(below_iv, ge_iv)
            return carry

        def rnd(_, carry):
            @pl.when(st[0, :][0] == 0)
            def _():
                lax.fori_loop(0, NVISIT, visit, jnp.int32(0))
            return carry

        lax.fori_loop(0, 16, visit, jnp.int32(0))
        lax.fori_loop(0, NROUND, rnd, jnp.int32(0))
        pltpu.sync_copy(out_v, out_h)


_sc_nms = pl.kernel(
    _nms_body,
    out_type=jax.ShapeDtypeStruct((16, LANES), jnp.float32),
    mesh=plsc.VectorSubcoreMesh(core_axis_name="c", subcore_axis_name="s"),
    scratch_types=[
        pltpu.VMEM((ROWS, LANES), jnp.float32),
        pltpu.VMEM_SHARED((4 * ROWS, LANES), jnp.float32),
        pltpu.VMEM((1, LANES), jnp.float32),
        pltpu.VMEM((NBLK, LANES), jnp.float32),
        pltpu.VMEM((NBLK, LANES), jnp.int32),
        pltpu.VMEM((16, LANES), jnp.float32),
        pltpu.VMEM((8, LANES), jnp.float32),
        pltpu.VMEM((2, LANES), jnp.int32),
        pltpu.SemaphoreType.DMA,
    ],
)


@jax.jit
def kernel(boxes, scores):
    npad = PAD - N_BOXES
    s = jnp.concatenate(
        [scores, jnp.full((npad,), NEG, jnp.float32)]).reshape(ROWS, LANES)
    b = jnp.concatenate([boxes, jnp.zeros((npad, 4), jnp.float32)])
    box = b.reshape(4 * ROWS, LANES)
    out = _sc_nms(s, box)
    return out[:TOPK, :5]
